# baseline (speedup 1.0000x reference)
import jax
import jax.numpy as jnp
from jax.experimental import pallas as pl
from jax.experimental.pallas import tpu as pltpu

N0 = 4096
N1 = 4096
C_IN = 256
C_OUT = 256
TILE_M = 512


def _proj_body(x0_ref, w_ref, xw_ref):
    xw_ref[...] = jnp.dot(
        x0_ref[...], w_ref[...], preferred_element_type=jnp.float32
    )


def _main_body(b_ref, xw_ref, out_ref):
    acc = jnp.dot(b_ref[...], xw_ref[...], preferred_element_type=jnp.float32)
    out_ref[...] = jnp.where(acc > 0, acc, jnp.exp(jnp.minimum(acc, 0.0)) - 1.0)


def kernel(x_0, neighborhood_0_to_1, W):
    xw = pl.pallas_call(
        _proj_body,
        out_shape=jax.ShapeDtypeStruct((N0, C_OUT), jnp.float32),
    )(x_0, W)
    return pl.pallas_call(
        _main_body,
        grid=(N1 // TILE_M,),
        in_specs=[
            pl.BlockSpec((TILE_M, N0), lambda i: (i, 0)),
            pl.BlockSpec((N0, C_OUT), lambda i: (0, 0)),
        ],
        out_specs=pl.BlockSpec((TILE_M, C_OUT), lambda i: (i, 0)),
        out_shape=jax.ShapeDtypeStruct((N1, C_OUT), jnp.float32),
        compiler_params=pltpu.CompilerParams(
            dimension_semantics=("parallel",),
        ),
    )(neighborhood_0_to_1, xw)


# speedup vs baseline: 1.0128x; 1.0128x over previous
import jax
import jax.numpy as jnp
from jax.experimental import pallas as pl
from jax.experimental.pallas import tpu as pltpu

N0 = 4096
N1 = 4096
C_IN = 256
C_OUT = 256
TILE_M = 256


def _fused_body(x0_ref, b_ref, w_ref, out_ref, xw_ref):
    @pl.when(pl.program_id(0) == 0)
    def _():
        xw_ref[...] = jnp.dot(
            x0_ref[...], w_ref[...], preferred_element_type=jnp.float32
        )

    acc = jnp.dot(b_ref[...], xw_ref[...], preferred_element_type=jnp.float32)
    out_ref[...] = jnp.where(acc > 0, acc, jnp.exp(jnp.minimum(acc, 0.0)) - 1.0)


def kernel(x_0, neighborhood_0_to_1, W):
    grid = (N1 // TILE_M,)
    return pl.pallas_call(
        _fused_body,
        grid=grid,
        in_specs=[
            pl.BlockSpec((N0, C_IN), lambda i: (0, 0)),
            pl.BlockSpec((TILE_M, N0), lambda i: (i, 0)),
            pl.BlockSpec((C_IN, C_OUT), lambda i: (0, 0)),
        ],
        out_specs=pl.BlockSpec((TILE_M, C_OUT), lambda i: (i, 0)),
        out_shape=jax.ShapeDtypeStruct((N1, C_OUT), jnp.float32),
        scratch_shapes=[pltpu.VMEM((N0, C_OUT), jnp.float32)],
    )(x_0, neighborhood_0_to_1, W)


# speedup vs baseline: 1.1226x; 1.1084x over previous
import jax
import jax.numpy as jnp
from jax.experimental import pallas as pl
from jax.experimental.pallas import tpu as pltpu

N0 = 4096
N1 = 4096
C_IN = 256
C_OUT = 256
TILE_M = 1024


def _fused_body(x0_ref, b_ref, w_ref, out_ref, xw_ref):
    @pl.when(pl.program_id(0) == 0)
    def _():
        xw_ref[...] = jnp.dot(
            x0_ref[...], w_ref[...], preferred_element_type=jnp.float32
        )

    acc = jnp.dot(b_ref[...], xw_ref[...], preferred_element_type=jnp.float32)
    out_ref[...] = jnp.where(acc > 0, acc, jnp.exp(jnp.minimum(acc, 0.0)) - 1.0)


def kernel(x_0, neighborhood_0_to_1, W):
    grid = (N1 // TILE_M,)
    return pl.pallas_call(
        _fused_body,
        grid=grid,
        in_specs=[
            pl.BlockSpec((N0, C_IN), lambda i: (0, 0)),
            pl.BlockSpec((TILE_M, N0), lambda i: (i, 0)),
            pl.BlockSpec((C_IN, C_OUT), lambda i: (0, 0)),
        ],
        out_specs=pl.BlockSpec((TILE_M, C_OUT), lambda i: (i, 0)),
        out_shape=jax.ShapeDtypeStruct((N1, C_OUT), jnp.float32),
        scratch_shapes=[pltpu.VMEM((N0, C_OUT), jnp.float32)],
    )(x_0, neighborhood_0_to_1, W)


# speedup vs baseline: 1.1629x; 1.0359x over previous
import jax
import jax.numpy as jnp
from jax.experimental import pallas as pl
from jax.experimental.pallas import tpu as pltpu

N0 = 4096
N1 = 4096
C_IN = 256
C_OUT = 256
TILE_M = 512


def _fused_body(x0_ref, b_ref, w_ref, out_ref, xw_ref):
    @pl.when(pl.program_id(0) == 0)
    def _():
        xw_ref[...] = jnp.dot(
            x0_ref[...], w_ref[...], preferred_element_type=jnp.float32
        ).astype(jnp.bfloat16)

    acc = jnp.dot(
        b_ref[...].astype(jnp.bfloat16),
        xw_ref[...],
        preferred_element_type=jnp.float32,
    )
    out_ref[...] = jnp.where(acc > 0, acc, jnp.exp(jnp.minimum(acc, 0.0)) - 1.0)


def kernel(x_0, neighborhood_0_to_1, W):
    grid = (N1 // TILE_M,)
    return pl.pallas_call(
        _fused_body,
        grid=grid,
        in_specs=[
            pl.BlockSpec((N0, C_IN), lambda i: (0, 0)),
            pl.BlockSpec((TILE_M, N0), lambda i: (i, 0)),
            pl.BlockSpec((C_IN, C_OUT), lambda i: (0, 0)),
        ],
        out_specs=pl.BlockSpec((TILE_M, C_OUT), lambda i: (i, 0)),
        out_shape=jax.ShapeDtypeStruct((N1, C_OUT), jnp.float32),
        scratch_shapes=[pltpu.VMEM((N0, C_OUT), jnp.bfloat16)],
    )(x_0, neighborhood_0_to_1, W)
